# baseline (speedup 1.0000x reference)
import functools

import jax
import jax.numpy as jnp
from jax import lax
from jax.experimental import pallas as pl
from jax.experimental.pallas import tpu as pltpu

_K = 3
_STRIDE = 1
_PAD = 1


def _sample(x, offset):
    B, C, H, W = x.shape
    Ho, Wo = offset.shape[2], offset.shape[3]
    KK = _K * _K
    off = offset.reshape(B, KK, 2, Ho, Wo)
    ky, kx = jnp.meshgrid(jnp.arange(_K), jnp.arange(_K), indexing="ij")
    ky = ky.reshape(KK).astype(x.dtype)
    kx = kx.reshape(KK).astype(x.dtype)
    base_y = (jnp.arange(Ho, dtype=x.dtype) * _STRIDE - _PAD)[None, :, None] + ky[:, None, None]
    base_x = (jnp.arange(Wo, dtype=x.dtype) * _STRIDE - _PAD)[None, None, :] + kx[:, None, None]
    py = off[:, :, 0] + base_y
    px = off[:, :, 1] + base_x

    def sample_one(img, py1, px1):
        y0f = jnp.floor(py1)
        x0f = jnp.floor(px1)
        wy1 = py1 - y0f
        wx1 = px1 - x0f
        wy0 = 1.0 - wy1
        wx0 = 1.0 - wx1
        y0 = y0f.astype(jnp.int32)
        x0 = x0f.astype(jnp.int32)
        y1 = y0 + 1
        x1 = x0 + 1

        def gather(yi, xi):
            valid = (yi >= 0) & (yi < H) & (xi >= 0) & (xi < W)
            yc = jnp.clip(yi, 0, H - 1)
            xc = jnp.clip(xi, 0, W - 1)
            v = img[:, yc, xc]
            return v * valid[None].astype(img.dtype)

        out = (wy0 * wx0)[None] * gather(y0, x0)
        out = out + (wy0 * wx1)[None] * gather(y0, x1)
        out = out + (wy1 * wx0)[None] * gather(y1, x0)
        out = out + (wy1 * wx1)[None] * gather(y1, x1)
        return out

    return jax.vmap(sample_one)(x, py, px)


def _matmul_body(w_ref, s_ref, o_ref):
    o_ref[0] = jnp.dot(
        w_ref[...],
        s_ref[0],
        preferred_element_type=jnp.float32,
        precision=lax.Precision.HIGHEST,
    )


@functools.partial(jax.jit, static_argnames=())
def _deform_contract(sampled_flat, w_flat):
    B, CK, HW = sampled_flat.shape
    O = w_flat.shape[0]
    BLK = 512
    grid = (B, HW // BLK)
    return pl.pallas_call(
        _matmul_body,
        grid=grid,
        in_specs=[
            pl.BlockSpec((O, CK), lambda b, t: (0, 0)),
            pl.BlockSpec((1, CK, BLK), lambda b, t: (b, 0, t)),
        ],
        out_specs=pl.BlockSpec((1, O, BLK), lambda b, t: (b, 0, t)),
        out_shape=jax.ShapeDtypeStruct((B, O, HW), jnp.float32),
        compiler_params=pltpu.CompilerParams(
            dimension_semantics=("parallel", "parallel")
        ),
    )(w_flat, sampled_flat)


def kernel(x, w_off, w_def):
    B, C, H, W = x.shape
    offset = lax.conv_general_dilated(
        x, w_off, window_strides=(_STRIDE, _STRIDE),
        padding=[(_PAD, _PAD), (_PAD, _PAD)],
        dimension_numbers=("NCHW", "OIHW", "NCHW"),
    )
    Ho, Wo = offset.shape[2], offset.shape[3]
    KK = _K * _K
    sampled = _sample(x, offset)
    sampled_flat = sampled.reshape(B, C * KK, Ho * Wo)
    w_flat = w_def.reshape(w_def.shape[0], C * KK)
    out = _deform_contract(sampled_flat, w_flat)
    return out.reshape(B, w_def.shape[0], Ho, Wo)


# speedup vs baseline: 1.0781x; 1.0781x over previous
import functools

import jax
import jax.numpy as jnp
from jax import lax
from jax.experimental import pallas as pl
from jax.experimental.pallas import tpu as pltpu

_K = 3
_STRIDE = 1
_PAD = 1


def _sample(x, offset):
    B, C, H, W = x.shape
    Ho, Wo = offset.shape[2], offset.shape[3]
    KK = _K * _K
    off = offset.reshape(B, KK, 2, Ho, Wo)
    ky, kx = jnp.meshgrid(jnp.arange(_K), jnp.arange(_K), indexing="ij")
    ky = ky.reshape(KK).astype(x.dtype)
    kx = kx.reshape(KK).astype(x.dtype)
    base_y = (jnp.arange(Ho, dtype=x.dtype) * _STRIDE - _PAD)[None, :, None] + ky[:, None, None]
    base_x = (jnp.arange(Wo, dtype=x.dtype) * _STRIDE - _PAD)[None, None, :] + kx[:, None, None]
    py = off[:, :, 0] + base_y
    px = off[:, :, 1] + base_x

    y0f = jnp.floor(py)
    x0f = jnp.floor(px)
    wy1 = py - y0f
    wx1 = px - x0f
    wy0 = 1.0 - wy1
    wx0 = 1.0 - wx1
    y0 = y0f.astype(jnp.int32)
    x0 = x0f.astype(jnp.int32)

    x_flat = x.reshape(B, C, H * W)
    P = KK * Ho * Wo
    out = jnp.zeros((B, C, P), x.dtype)
    for dy, wy in ((0, wy0), (1, wy1)):
        for dx, wx in ((0, wx0), (1, wx1)):
            yi = y0 + dy
            xi = x0 + dx
            valid = (yi >= 0) & (yi < H) & (xi >= 0) & (xi < W)
            flat = jnp.clip(yi, 0, H - 1) * W + jnp.clip(xi, 0, W - 1)
            v = jnp.take_along_axis(
                x_flat, flat.reshape(B, 1, P), axis=2
            )
            w = (wy * wx * valid.astype(x.dtype)).reshape(B, 1, P)
            out = out + w * v
    return out.reshape(B, C, KK, Ho, Wo)


def _matmul_body(w_ref, s_ref, o_ref):
    o_ref[0] = jnp.dot(
        w_ref[...],
        s_ref[0],
        preferred_element_type=jnp.float32,
    )


@functools.partial(jax.jit, static_argnames=())
def _deform_contract(sampled_flat, w_flat):
    B, CK, HW = sampled_flat.shape
    O = w_flat.shape[0]
    BLK = 512
    grid = (B, HW // BLK)
    return pl.pallas_call(
        _matmul_body,
        grid=grid,
        in_specs=[
            pl.BlockSpec((O, CK), lambda b, t: (0, 0)),
            pl.BlockSpec((1, CK, BLK), lambda b, t: (b, 0, t)),
        ],
        out_specs=pl.BlockSpec((1, O, BLK), lambda b, t: (b, 0, t)),
        out_shape=jax.ShapeDtypeStruct((B, O, HW), jnp.float32),
        compiler_params=pltpu.CompilerParams(
            dimension_semantics=("parallel", "parallel")
        ),
    )(w_flat, sampled_flat)


def kernel(x, w_off, w_def):
    B, C, H, W = x.shape
    offset = lax.conv_general_dilated(
        x, w_off, window_strides=(_STRIDE, _STRIDE),
        padding=[(_PAD, _PAD), (_PAD, _PAD)],
        dimension_numbers=("NCHW", "OIHW", "NCHW"),
    )
    Ho, Wo = offset.shape[2], offset.shape[3]
    KK = _K * _K
    sampled = _sample(x, offset)
    sampled_flat = sampled.reshape(B, C * KK, Ho * Wo)
    w_flat = w_def.reshape(w_def.shape[0], C * KK)
    out = _deform_contract(sampled_flat, w_flat)
    return out.reshape(B, w_def.shape[0], Ho, Wo)


# speedup vs baseline: 1.0837x; 1.0052x over previous
import jax
import jax.numpy as jnp
from jax import lax
from jax.experimental import pallas as pl
from jax.experimental.pallas import tpu as pltpu

_K = 3
_STRIDE = 1
_PAD = 1


def _sample_cl(x, offset):
    B, C, H, W = x.shape
    Ho, Wo = offset.shape[2], offset.shape[3]
    KK = _K * _K
    HW = Ho * Wo
    off = offset.reshape(B, KK, 2, Ho, Wo)
    ky, kx = jnp.meshgrid(jnp.arange(_K), jnp.arange(_K), indexing="ij")
    ky = ky.reshape(KK).astype(x.dtype)
    kx = kx.reshape(KK).astype(x.dtype)
    base_y = (jnp.arange(Ho, dtype=x.dtype) * _STRIDE - _PAD)[None, :, None] + ky[:, None, None]
    base_x = (jnp.arange(Wo, dtype=x.dtype) * _STRIDE - _PAD)[None, None, :] + kx[:, None, None]
    py = (off[:, :, 0] + base_y).transpose(0, 2, 3, 1).reshape(B, HW * KK)
    px = (off[:, :, 1] + base_x).transpose(0, 2, 3, 1).reshape(B, HW * KK)

    y0f = jnp.floor(py)
    x0f = jnp.floor(px)
    wy1 = py - y0f
    wx1 = px - x0f
    wy0 = 1.0 - wy1
    wx0 = 1.0 - wx1
    y0 = y0f.astype(jnp.int32)
    x0 = x0f.astype(jnp.int32)

    x_cl = x.transpose(0, 2, 3, 1).reshape(B, H * W, C)
    acc = jnp.zeros((B, HW * KK, C), x.dtype)
    for dy, wy in ((0, wy0), (1, wy1)):
        for dx, wx in ((0, wx0), (1, wx1)):
            yi = y0 + dy
            xi = x0 + dx
            valid = (yi >= 0) & (yi < H) & (xi >= 0) & (xi < W)
            flat = jnp.clip(yi, 0, H - 1) * W + jnp.clip(xi, 0, W - 1)
            v = jnp.take_along_axis(x_cl, flat[:, :, None], axis=1)
            w = (wy * wx * valid.astype(x.dtype))[:, :, None]
            acc = acc + w * v
    return acc.reshape(B, HW, KK * C)


def _matmul_body(s_ref, w_ref, o_ref):
    o_ref[0] = jnp.dot(
        s_ref[0], w_ref[...], preferred_element_type=jnp.float32
    )


def _deform_contract(sampled, w2):
    B, HW, KC = sampled.shape
    O = w2.shape[1]
    BLK = 512
    grid = (B, HW // BLK)
    return pl.pallas_call(
        _matmul_body,
        grid=grid,
        in_specs=[
            pl.BlockSpec((1, BLK, KC), lambda b, t: (b, t, 0)),
            pl.BlockSpec((KC, O), lambda b, t: (0, 0)),
        ],
        out_specs=pl.BlockSpec((1, BLK, O), lambda b, t: (b, t, 0)),
        out_shape=jax.ShapeDtypeStruct((B, HW, O), jnp.float32),
        compiler_params=pltpu.CompilerParams(
            dimension_semantics=("parallel", "parallel")
        ),
    )(sampled, w2)


def kernel(x, w_off, w_def):
    B, C, H, W = x.shape
    offset = lax.conv_general_dilated(
        x, w_off, window_strides=(_STRIDE, _STRIDE),
        padding=[(_PAD, _PAD), (_PAD, _PAD)],
        dimension_numbers=("NCHW", "OIHW", "NCHW"),
    )
    Ho, Wo = offset.shape[2], offset.shape[3]
    KK = _K * _K
    O = w_def.shape[0]
    sampled = _sample_cl(x, offset)
    w2 = w_def.reshape(O, C, KK).transpose(2, 1, 0).reshape(KK * C, O)
    out = _deform_contract(sampled, w2)
    return out.transpose(0, 2, 1).reshape(B, O, Ho, Wo)
